# baseline (speedup 1.0000x reference)
import functools

import jax
import jax.numpy as jnp
from jax import lax
from jax.experimental import pallas as pl
from jax.experimental.pallas import tpu as pltpu
from jax.experimental.pallas import tpu_sc as plsc

B = 128
N = 8192
O = 8192
K = 16
L = 16
NC, NS = 2, 16
NW = NC * NS

N_OC = 8
N_BC = NW // N_OC
OC = O // N_OC
HB = B // 2
NP = HB // N_BC
GP = 4
NG = NP // GP
NJ = OC // L

_mesh = plsc.VectorSubcoreMesh(core_axis_name="c", subcore_axis_name="s")
_PAIR = plsc.PackFormat.INTERLEAVED


@functools.partial(
    pl.kernel,
    out_type=jax.ShapeDtypeStruct((B, O), jnp.float32),
    mesh=_mesh,
    compiler_params=pltpu.CompilerParams(needs_layout_passes=False),
    scratch_types=(
        [
            pltpu.VMEM((K // 2, OC), jnp.int32),
            pltpu.VMEM((K, OC), jnp.int32),
        ]
        + [pltpu.VMEM((GP, OC), jnp.float32) for _ in range(4)]
        + [pltpu.VMEM((N,), jnp.int32) for _ in range(2 * GP)]
        + [
            pltpu.SemaphoreType.DMA,
            pltpu.SemaphoreType.DMA,
            pltpu.SemaphoreType.DMA,
        ]
    ),
)
def _sc_interconnect(
    xp_hbm, c_hbm, idxp_hbm, out_hbm,
    idx_v, w_v, oa0, ob0, oa1, ob1, *rest
):
    xp_vs, (sem_x, sem_io, sem_out) = rest[: 2 * GP], rest[2 * GP :]
    o_bufs = ((oa0, ob0), (oa1, ob1))
    cid = lax.axis_index("c")
    sid = lax.axis_index("s")
    wid = cid * NS + sid
    oc = wid % N_OC
    bc = wid // N_OC
    o0 = oc * OC
    p_base = bc * NP

    cp_i = pltpu.async_copy(idxp_hbm.at[:, pl.ds(o0, OC)], idx_v, sem_io)
    cp_w = pltpu.async_copy(c_hbm.at[:, pl.ds(o0, OC)], w_v, sem_io)

    def start_group(g):
        bufs = xp_vs[(g % 2) * GP : (g % 2) * GP + GP]
        return [
            pltpu.async_copy(
                xp_hbm.at[pl.ds((p_base + g * GP + p) * N, N)], bufs[p], sem_x
            )
            for p in range(GP)
        ]

    x_pending = start_group(0)

    cp_i.wait()
    cp_w.wait()

    @plsc.parallel_loop(0, NJ, unroll=2)
    def _softmax_body(j):
        off = j * L
        ws = [plsc.bitcast(w_v[k, pl.ds(off, L)], jnp.float32) for k in range(K)]
        es = [jnp.exp(w) for w in ws]
        ss = es
        while len(ss) > 1:
            ss = [ss[i] + ss[i + 1] for i in range(0, len(ss), 2)]
        inv = 1.0 / ss[0]
        for k in range(K):
            wf = es[k] * inv
            wd = plsc.pack(wf, wf, format=_PAIR)
            w_v[k, pl.ds(off, L)] = plsc.bitcast(wd, jnp.int32)

    out_pending = []
    for g in range(NG):
        for cp in x_pending:
            cp.wait()
        if g + 1 < NG:
            x_pending = start_group(g + 1)

        bufs = xp_vs[(g % 2) * GP : (g % 2) * GP + GP]
        oa, ob = o_bufs[g % 2]
        if g >= 2:
            for cp in out_pending[0]:
                cp.wait()
            out_pending = out_pending[1:]

        @plsc.parallel_loop(0, NJ, unroll=6)
        def _jbody(j):
            off = j * L
            idxs = []
            for kk in range(K // 2):
                pk = plsc.bitcast(idx_v[kk, pl.ds(off, L)], jnp.int16)
                i_even, i_odd = plsc.unpack(pk, format=_PAIR)
                idxs.append(i_even)
                idxs.append(i_odd)
            wds = [
                plsc.bitcast(w_v[k, pl.ds(off, L)], jnp.bfloat16)
                for k in range(K)
            ]
            for p in range(GP):
                prods = [
                    plsc.bitcast(plsc.load_gather(bufs[p], [idxs[k]]),
                                 jnp.bfloat16) * wds[k]
                    for k in range(K)
                ]
                pairs = [prods[i] + prods[i + 1] for i in range(0, K, 2)]
                accs_a = []
                accs_b = []
                for pr in pairs:
                    a, b = plsc.unpack(pr, format=_PAIR)
                    accs_a.append(a)
                    accs_b.append(b)
                while len(accs_a) > 1:
                    accs_a = [accs_a[i] + accs_a[i + 1]
                              for i in range(0, len(accs_a), 2)]
                    accs_b = [accs_b[i] + accs_b[i + 1]
                              for i in range(0, len(accs_b), 2)]
                oa[p, pl.ds(off, L)] = accs_a[0]
                ob[p, pl.ds(off, L)] = accs_b[0]

        row_a = p_base + g * GP
        out_pending.append([
            pltpu.async_copy(
                oa, out_hbm.at[pl.ds(row_a, GP), pl.ds(o0, OC)], sem_out
            ),
            pltpu.async_copy(
                ob, out_hbm.at[pl.ds(HB + row_a, GP), pl.ds(o0, OC)], sem_out
            ),
        ])

    for grp in out_pending:
        for cp in grp:
            cp.wait()


def kernel(x, top_c, top_indices):
    xb = jax.lax.bitcast_convert_type(
        x.astype(jnp.bfloat16), jnp.uint16
    ).astype(jnp.uint32)
    xp = (xb[:HB] | (xb[HB:] << 16)).astype(jnp.int32).reshape(-1)
    iu = top_indices.astype(jnp.uint32)
    idxp = (iu[0::2] | (iu[1::2] << 16)).astype(jnp.int32)
    c_i = jax.lax.bitcast_convert_type(top_c, jnp.int32)
    return _sc_interconnect(xp, c_i, idxp)


# speedup vs baseline: 1.0956x; 1.0956x over previous
import functools

import jax
import jax.numpy as jnp
from jax import lax
from jax.experimental import pallas as pl
from jax.experimental.pallas import tpu as pltpu
from jax.experimental.pallas import tpu_sc as plsc

B = 128
N = 8192
O = 8192
K = 16
L = 16
NC, NS = 2, 16
NW = NC * NS

N_OC = 8
N_BC = NW // N_OC
OC = O // N_OC
HB = B // 2
NP = HB // N_BC
GP = 4
NG = NP // GP
NJ = OC // L

_mesh = plsc.VectorSubcoreMesh(core_axis_name="c", subcore_axis_name="s")
_PAIR = plsc.PackFormat.INTERLEAVED


@functools.partial(
    pl.kernel,
    out_type=jax.ShapeDtypeStruct((B, O), jnp.float32),
    mesh=_mesh,
    compiler_params=pltpu.CompilerParams(needs_layout_passes=False),
    scratch_types=(
        [
            pltpu.VMEM((K // 2, OC), jnp.int32),
            pltpu.VMEM((K, OC), jnp.int32),
        ]
        + [pltpu.VMEM((GP, OC), jnp.float32) for _ in range(4)]
        + [pltpu.VMEM((N,), jnp.int32) for _ in range(2 * GP)]
        + [
            pltpu.SemaphoreType.DMA,
            pltpu.SemaphoreType.DMA,
            pltpu.SemaphoreType.DMA,
        ]
    ),
)
def _sc_interconnect(
    xp_hbm, c_hbm, idxp_hbm, out_hbm,
    idx_v, w_v, oa0, ob0, oa1, ob1, *rest
):
    xp_vs, (sem_x, sem_io, sem_out) = rest[: 2 * GP], rest[2 * GP :]
    o_bufs = ((oa0, ob0), (oa1, ob1))
    cid = lax.axis_index("c")
    sid = lax.axis_index("s")
    wid = cid * NS + sid
    oc = wid % N_OC
    bc = wid // N_OC
    o0 = oc * OC
    p_base = bc * NP

    cp_i = pltpu.async_copy(idxp_hbm.at[:, pl.ds(o0, OC)], idx_v, sem_io)
    cp_w = pltpu.async_copy(c_hbm.at[:, pl.ds(o0, OC)], w_v, sem_io)

    def start_group(g):
        bufs = xp_vs[(g % 2) * GP : (g % 2) * GP + GP]
        return [
            pltpu.async_copy(
                xp_hbm.at[pl.ds((p_base + g * GP + p) * N, N)], bufs[p], sem_x
            )
            for p in range(GP)
        ]

    x_pending = start_group(0)

    cp_i.wait()
    cp_w.wait()

    @plsc.parallel_loop(0, NJ, unroll=2)
    def _softmax_body(j):
        off = j * L
        ws = [plsc.bitcast(w_v[k, pl.ds(off, L)], jnp.float32) for k in range(K)]
        es = [jnp.exp(w) for w in ws]
        ss = es
        while len(ss) > 1:
            ss = [ss[i] + ss[i + 1] for i in range(0, len(ss), 2)]
        inv = 1.0 / ss[0]
        for k in range(K):
            wf = es[k] * inv
            wd = plsc.pack(wf, wf, format=_PAIR)
            w_v[k, pl.ds(off, L)] = plsc.bitcast(wd, jnp.int32)

    out_pending = []
    for g in range(NG):
        for cp in x_pending:
            cp.wait()
        if g + 1 < NG:
            x_pending = start_group(g + 1)

        bufs = xp_vs[(g % 2) * GP : (g % 2) * GP + GP]
        oa, ob = o_bufs[g % 2]
        if g >= 2:
            for cp in out_pending[0]:
                cp.wait()
            out_pending = out_pending[1:]

        @plsc.parallel_loop(0, NJ, unroll=4)
        def _jbody(j):
            off = j * L
            idxs = []
            for kk in range(K // 2):
                pk = plsc.bitcast(idx_v[kk, pl.ds(off, L)], jnp.int16)
                i_even, i_odd = plsc.unpack(pk, format=_PAIR)
                idxs.append(i_even)
                idxs.append(i_odd)
            wds = [
                plsc.bitcast(w_v[k, pl.ds(off, L)], jnp.bfloat16)
                for k in range(K)
            ]
            for p in range(GP):
                prods = [
                    plsc.bitcast(plsc.load_gather(bufs[p], [idxs[k]]),
                                 jnp.bfloat16) * wds[k]
                    for k in range(K)
                ]
                pairs = [prods[i] + prods[i + 1] for i in range(0, K, 2)]
                accs_a = []
                accs_b = []
                for pr in pairs:
                    a, b = plsc.unpack(pr, format=_PAIR)
                    accs_a.append(a)
                    accs_b.append(b)
                while len(accs_a) > 1:
                    accs_a = [accs_a[i] + accs_a[i + 1]
                              for i in range(0, len(accs_a), 2)]
                    accs_b = [accs_b[i] + accs_b[i + 1]
                              for i in range(0, len(accs_b), 2)]
                oa[p, pl.ds(off, L)] = accs_a[0]
                ob[p, pl.ds(off, L)] = accs_b[0]

        row_a = p_base + g * GP
        out_pending.append([
            pltpu.async_copy(
                oa, out_hbm.at[pl.ds(row_a, GP), pl.ds(o0, OC)], sem_out
            ),
            pltpu.async_copy(
                ob, out_hbm.at[pl.ds(HB + row_a, GP), pl.ds(o0, OC)], sem_out
            ),
        ])

    for grp in out_pending:
        for cp in grp:
            cp.wait()


def kernel(x, top_c, top_indices):
    xb = jax.lax.bitcast_convert_type(
        x.astype(jnp.bfloat16), jnp.uint16
    ).astype(jnp.uint32)
    xp = (xb[:HB] | (xb[HB:] << 16)).astype(jnp.int32).reshape(-1)
    iu = top_indices.astype(jnp.uint32)
    idxp = (iu[0::2] | (iu[1::2] << 16)).astype(jnp.int32)
    c_i = jax.lax.bitcast_convert_type(top_c, jnp.int32)
    return _sc_interconnect(xp, c_i, idxp)
